# Initial kernel scaffold; baseline (speedup 1.0000x reference)
import jax
import jax.numpy as jnp
from jax.experimental import pallas as pl


def kernel(x, W):
    raise NotImplementedError("write your pallas kernel here")


# speedup vs baseline: 1.4682x; 1.4682x over previous
import functools

import jax
import jax.numpy as jnp
from jax import lax
from jax.experimental import pallas as pl
from jax.experimental.pallas import tpu as pltpu
from jax.experimental.pallas import tpu_sc as plsc

NC, NS, LANES = 2, 16, 16
NW = NC * NS
B = 4096 * 200
D = 32
BPW = B // NW
SUB = 128
CHUNK = 1024
NSUB = CHUNK // SUB
NCHUNK = BPW // CHUNK


def _build_sc_kernel():
  mesh = plsc.VectorSubcoreMesh(core_axis_name="c", subcore_axis_name="s")

  @functools.partial(
      pl.kernel,
      out_type=jax.ShapeDtypeStruct((B, D), jnp.float32),
      mesh=mesh,
      compiler_params=pltpu.CompilerParams(
          needs_layout_passes=False, use_tc_tiling_on_sc=False),
      scratch_types=[
          pltpu.VMEM((NSUB, SUB), jnp.int32),
          pltpu.VMEM((CHUNK, D), jnp.float32),
          pltpu.SemaphoreType.DMA,
      ],
  )
  def k(idx_hbm, table_hbm, out_hbm, idx_v, rows_v, sem):
    wid = lax.axis_index("s") * NC + lax.axis_index("c")

    def chunk_body(c, carry):
      gbase = pl.multiple_of(wid * (BPW // SUB) + c * NSUB, 8)
      base = pl.multiple_of(wid * BPW + c * CHUNK, 8)

      pltpu.sync_copy(idx_hbm.at[pl.ds(gbase, NSUB)], idx_v)

      descs = []
      for j in range(NSUB):
        descs.append(
            pltpu.async_copy(
                table_hbm.at[idx_v.at[j]],
                rows_v.at[pl.ds(j * SUB, SUB)],
                sem,
            ))
      for d in descs:
        d.wait()

      acc = idx_v[0, pl.ds(0, LANES)] <= 0
      for t in range(1, CHUNK // LANES):
        j, i = divmod(t, SUB // LANES)
        acc = jnp.logical_or(acc, idx_v[j, pl.ds(i * LANES, LANES)] <= 0)
      nbad = plsc.all_reduce_population_count(acc)[0]

      @pl.when(nbad > 0)
      def _fixup():
        for j in range(NSUB):
          def grp_body(g, _, j=j):
            v = idx_v[j, pl.ds(g * LANES, LANES)]
            gbad = plsc.all_reduce_population_count(v <= 0)[0]

            @pl.when(gbad > 0)
            def _fix_group():
              for l in range(LANES):
                m = (v[l] > 0).astype(jnp.float32)
                rr = j * SUB + g * LANES + l
                rows_v[rr, pl.ds(0, LANES)] = (
                    rows_v[rr, pl.ds(0, LANES)] * m)
                rows_v[rr, pl.ds(LANES, LANES)] = (
                    rows_v[rr, pl.ds(LANES, LANES)] * m)

            return 0
          lax.fori_loop(0, SUB // LANES, grp_body, 0)

      pltpu.sync_copy(rows_v, out_hbm.at[pl.ds(base, CHUNK)])
      return carry

    lax.fori_loop(0, NCHUNK, chunk_body, 0)

  return k


_gather = _build_sc_kernel()


def kernel(x, W):
  idx = x.reshape(B // SUB, SUB)
  out = _gather(idx, W)
  return out.reshape(x.shape[0], x.shape[1], D)


# speedup vs baseline: 1.5048x; 1.0249x over previous
import functools

import jax
import jax.numpy as jnp
from jax import lax
from jax.experimental import pallas as pl
from jax.experimental.pallas import tpu as pltpu
from jax.experimental.pallas import tpu_sc as plsc

NC, NS, LANES = 2, 16, 16
NW = NC * NS
B = 4096 * 200
D = 32
BPW = B // NW
SUB = 128
CHUNK = 1280
NSUB = CHUNK // SUB
NCHUNK = BPW // CHUNK


def _build_sc_kernel():
  mesh = plsc.VectorSubcoreMesh(core_axis_name="c", subcore_axis_name="s")

  @functools.partial(
      pl.kernel,
      out_type=jax.ShapeDtypeStruct((B, D), jnp.float32),
      mesh=mesh,
      compiler_params=pltpu.CompilerParams(
          needs_layout_passes=False, use_tc_tiling_on_sc=False),
      scratch_types=[
          pltpu.VMEM((NSUB, SUB), jnp.int32),
          pltpu.VMEM((NSUB, SUB), jnp.int32),
          pltpu.VMEM((CHUNK, D), jnp.float32),
          pltpu.VMEM((CHUNK, D), jnp.float32),
          pltpu.SemaphoreType.DMA,
          pltpu.SemaphoreType.DMA,
          pltpu.SemaphoreType.DMA,
          pltpu.SemaphoreType.DMA,
      ],
  )
  def k(idx_hbm, table_hbm, out_hbm, idx0, idx1, rows0, rows1,
        gsem, isem, s0, s1):
    wid = lax.axis_index("s") * NC + lax.axis_index("c")
    grow0 = wid * (BPW // SUB)
    row0 = wid * BPW

    def idx_copy(c, idx_b):
      gb = grow0 + c * NSUB
      return pltpu.make_async_copy(
          idx_hbm.at[pl.ds(gb, NSUB)], idx_b, isem)

    def store_copy(c, rows_b, ssem):
      base = pl.multiple_of(row0 + c * CHUNK, 8)
      return pltpu.make_async_copy(
          rows_b, out_hbm.at[pl.ds(base, CHUNK)], ssem)

    def scan_bad(idx_b):
      acc = idx_b[0, pl.ds(0, LANES)] <= 0
      for t in range(1, CHUNK // LANES):
        j, i = divmod(t, SUB // LANES)
        acc = jnp.logical_or(acc, idx_b[j, pl.ds(i * LANES, LANES)] <= 0)
      return plsc.all_reduce_population_count(acc)[0]

    def fixup(nbad, idx_b, rows_b):
      @pl.when(nbad > 0)
      def _fixup():
        for j in range(NSUB):
          def grp_body(g, _, j=j):
            v = idx_b[j, pl.ds(g * LANES, LANES)]
            gbad = plsc.all_reduce_population_count(v <= 0)[0]

            @pl.when(gbad > 0)
            def _fix_group():
              for l in range(LANES):
                m = (v[l] > 0).astype(jnp.float32)
                rr = j * SUB + g * LANES + l
                rows_b[rr, pl.ds(0, LANES)] = (
                    rows_b[rr, pl.ds(0, LANES)] * m)
                rows_b[rr, pl.ds(LANES, LANES)] = (
                    rows_b[rr, pl.ds(LANES, LANES)] * m)

            return 0
          lax.fori_loop(0, SUB // LANES, grp_body, 0)

    def half(c, idx_a, rows_a, ssem_a, idx_b):
      @pl.when(c >= 2)
      def _ws():
        store_copy(c - 2, rows_a, ssem_a).wait()

      descs = []
      for j in range(NSUB):
        descs.append(
            pltpu.async_copy(
                table_hbm.at[idx_a.at[j]],
                rows_a.at[pl.ds(j * SUB, SUB)], gsem))

      @pl.when(c + 1 < NCHUNK)
      def _pf():
        idx_copy(c + 1, idx_b).start()

      nbad = scan_bad(idx_a)
      for dsc in descs:
        dsc.wait()
      fixup(nbad, idx_a, rows_a)
      store_copy(c, rows_a, ssem_a).start()

      @pl.when(c + 1 < NCHUNK)
      def _pfw():
        idx_copy(c + 1, idx_b).wait()

    idx_copy(0, idx0).start()
    idx_copy(0, idx0).wait()

    def body(i, carry):
      c0 = i * 2
      half(c0, idx0, rows0, s0, idx1)
      half(c0 + 1, idx1, rows1, s1, idx0)
      return carry

    lax.fori_loop(0, NCHUNK // 2, body, 0)
    store_copy(NCHUNK - 2, rows0, s0).wait()
    store_copy(NCHUNK - 1, rows1, s1).wait()

  return k


_gather = _build_sc_kernel()


def kernel(x, W):
  idx = x.reshape(B // SUB, SUB)
  out = _gather(idx, W)
  return out.reshape(x.shape[0], x.shape[1], D)
